# baseline (speedup 1.0000x reference)
import jax
import jax.numpy as jnp
from jax import lax
from jax.experimental import pallas as pl
from jax.experimental.pallas import tpu as pltpu
from jax.experimental.pallas import tpu_sc as plsc

B = 4096
S = 200
D = 64
C_OUT = 2
NC, NS = 2, 16
NW = NC * NS
BPW = B // NW
P = 16
R = 8
G = S // R
GSZ = P * R
NCH = BPW // P
LG = D // 16


def _pool_body(xarr, table, out, idx_v, acc_v, obuf, sem):
    wid = lax.axis_index("s") * NC + lax.axis_index("c")
    pltpu.sync_copy(xarr.at[wid], idx_v)
    zero = jnp.zeros((16,), jnp.float32)

    def chunk(c, carry):
        def zbody(r, _):
            for k in range(LG):
                acc_v[r, pl.ds(k * 16, 16)] = zero
            return 0

        lax.fori_loop(0, GSZ, zbody, 0)

        base = c * (G * GSZ)

        def fire(g, _):
            pltpu.async_copy(
                table.at[idx_v.at[pl.ds(base + g * GSZ, GSZ)]],
                acc_v,
                sem,
                add=True,
            )
            return 0

        lax.fori_loop(0, G, fire, 0)

        def drain(g, _):
            pltpu.make_async_copy(
                table.at[idx_v.at[pl.ds(base, GSZ)]], acc_v, sem
            ).wait()
            return 0

        lax.fori_loop(0, G, drain, 0)

        def red(p, _):
            for k in range(LG):
                v = acc_v[p * R, pl.ds(k * 16, 16)]
                for j in range(1, R):
                    v = v + acc_v[p * R + j, pl.ds(k * 16, 16)]
                obuf[c * P + p, pl.ds(k * 16, 16)] = v
            return 0

        lax.fori_loop(0, P, red, 0)
        return carry

    lax.fori_loop(0, NCH, chunk, 0)
    pltpu.sync_copy(obuf, out.at[pl.ds(wid * BPW, BPW)])


def _proj_body(p_ref, w_ref, b_ref, o_ref):
    acc = lax.dot_general(
        p_ref[...],
        w_ref[...],
        (((1,), (1,)), ((), ())),
        preferred_element_type=jnp.float32,
        precision=lax.Precision.HIGHEST,
    )
    o_ref[...] = acc * (1.0 / S) + b_ref[...]


def kernel(x, emb_table, W, b):
    xarr = (
        x.reshape(NW, NCH, P, G, R)
        .transpose(0, 1, 3, 2, 4)
        .reshape(NW, NCH * G * GSZ)
    )

    mesh = plsc.VectorSubcoreMesh(
        core_axis_name="c", subcore_axis_name="s", num_cores=NC, num_subcores=NS
    )
    pooled_sums = pl.kernel(
        _pool_body,
        out_type=jax.ShapeDtypeStruct((B, D), jnp.float32),
        mesh=mesh,
        compiler_params=pltpu.CompilerParams(use_tc_tiling_on_sc=False),
        scratch_types=[
            pltpu.VMEM((NCH * G * GSZ,), jnp.int32),
            pltpu.VMEM((GSZ, D), jnp.float32),
            pltpu.VMEM((BPW, D), jnp.float32),
            pltpu.SemaphoreType.DMA,
        ],
    )(xarr, emb_table)

    out = pl.pallas_call(
        _proj_body,
        out_shape=jax.ShapeDtypeStruct((B, C_OUT), jnp.float32),
    )(pooled_sums, W, b.reshape(1, C_OUT))
    return out


# speedup vs baseline: 1.0093x; 1.0093x over previous
import jax
import jax.numpy as jnp
from jax import lax
from jax.experimental import pallas as pl
from jax.experimental.pallas import tpu as pltpu
from jax.experimental.pallas import tpu_sc as plsc

B = 4096
S = 200
D = 64
C_OUT = 2
NC, NS = 2, 16
NW = NC * NS
BPW = B // NW
P = 16
R = 8
G = S // R
GSZ = P * R
NCH = BPW // P
LG = D // 16


def _pool_body(x_hbm, table, out, xrows, idx2, acc_v, obuf, sem):
    wid = lax.axis_index("s") * NC + lax.axis_index("c")
    pltpu.sync_copy(x_hbm.at[pl.ds(wid * BPW, BPW)], xrows)
    zero = jnp.zeros((16,), jnp.float32)
    lane = jax.lax.iota(jnp.int32, 16)
    lane_hi = lane >> 3
    lane_lo = lane & 7

    def chunk(c, carry):
        def bld_g(g, _):
            col = g * R + lane_lo

            def bld_p(p2, _):
                row = c * P + p2 * 2 + lane_hi
                v = plsc.load_gather(xrows, [row, col])
                idx2[pl.ds(g * GSZ + p2 * 16, 16)] = v
                return 0

            lax.fori_loop(0, P // 2, bld_p, 0)
            return 0

        lax.fori_loop(0, G, bld_g, 0)

        def zbody(r, _):
            for k in range(LG):
                acc_v[r, pl.ds(k * 16, 16)] = zero
            return 0

        lax.fori_loop(0, GSZ, zbody, 0)

        def fire(g, _):
            pltpu.async_copy(
                table.at[idx2.at[pl.ds(g * GSZ, GSZ)]],
                acc_v,
                sem,
                add=True,
            )
            return 0

        lax.fori_loop(0, G, fire, 0)

        def drain(g, _):
            pltpu.make_async_copy(
                table.at[idx2.at[pl.ds(0, GSZ)]], acc_v, sem
            ).wait()
            return 0

        lax.fori_loop(0, G, drain, 0)

        def red(p, _):
            for k in range(LG):
                v = acc_v[p * R, pl.ds(k * 16, 16)]
                for j in range(1, R):
                    v = v + acc_v[p * R + j, pl.ds(k * 16, 16)]
                obuf[c * P + p, pl.ds(k * 16, 16)] = v
            return 0

        lax.fori_loop(0, P, red, 0)
        return carry

    lax.fori_loop(0, NCH, chunk, 0)
    pltpu.sync_copy(obuf, out.at[pl.ds(wid * BPW, BPW)])


def _proj_body(p_ref, w_ref, b_ref, o_ref):
    acc = lax.dot_general(
        p_ref[...],
        w_ref[...],
        (((1,), (1,)), ((), ())),
        preferred_element_type=jnp.float32,
        precision=lax.Precision.HIGHEST,
    )
    o_ref[...] = acc * (1.0 / S) + b_ref[...]


def kernel(x, emb_table, W, b):
    mesh = plsc.VectorSubcoreMesh(
        core_axis_name="c", subcore_axis_name="s", num_cores=NC, num_subcores=NS
    )
    pooled_sums = pl.kernel(
        _pool_body,
        out_type=jax.ShapeDtypeStruct((B, D), jnp.float32),
        mesh=mesh,
        compiler_params=pltpu.CompilerParams(
            use_tc_tiling_on_sc=False, needs_layout_passes=False
        ),
        scratch_types=[
            pltpu.VMEM((BPW, S), jnp.int32),
            pltpu.VMEM((G * GSZ,), jnp.int32),
            pltpu.VMEM((GSZ, D), jnp.float32),
            pltpu.VMEM((BPW, D), jnp.float32),
            pltpu.SemaphoreType.DMA,
        ],
    )(x, emb_table)

    out = pl.pallas_call(
        _proj_body,
        out_shape=jax.ShapeDtypeStruct((B, C_OUT), jnp.float32),
    )(pooled_sums, W, b.reshape(1, C_OUT))
    return out
